# Initial kernel scaffold; baseline (speedup 1.0000x reference)
import jax
import jax.numpy as jnp
from jax.experimental import pallas as pl


def kernel(seq_len, table):
    raise NotImplementedError("write your pallas kernel here")


# speedup vs baseline: 1703.3073x; 1703.3073x over previous
import jax
import jax.numpy as jnp
from jax.experimental import pallas as pl
from jax.experimental.pallas import tpu as pltpu

_ROWS_PER_BLOCK = 8


def _bcast_row_kernel(idx_ref, tile_ref, out_ref):
    r = idx_ref[0] % _ROWS_PER_BLOCK
    row = tile_ref[pl.ds(r, 1), :]
    d = out_ref.shape[0]
    col = row.reshape(d, 1)
    out_ref[...] = jnp.broadcast_to(col, out_ref.shape)


def kernel(seq_len, table):
    n_rows, d_model = table.shape
    static_len = (n_rows + 1) // 2
    idx = jnp.clip(jnp.asarray(seq_len, jnp.int32) - 1, 0, n_rows - 1)
    idx = idx.reshape(1)
    grid_spec = pltpu.PrefetchScalarGridSpec(
        num_scalar_prefetch=1,
        grid=(1,),
        in_specs=[
            pl.BlockSpec(
                (_ROWS_PER_BLOCK, d_model),
                lambda i, idx_ref: (idx_ref[0] // _ROWS_PER_BLOCK, 0),
            )
        ],
        out_specs=pl.BlockSpec(
            (d_model, static_len), lambda i, idx_ref: (0, 0)
        ),
    )
    return pl.pallas_call(
        _bcast_row_kernel,
        grid_spec=grid_spec,
        out_shape=jax.ShapeDtypeStruct((d_model, static_len), table.dtype),
    )(idx, table)
